# Initial kernel scaffold; baseline (speedup 1.0000x reference)
import jax
import jax.numpy as jnp
from jax.experimental import pallas as pl


def kernel(attn_weights, attention_mask):
    raise NotImplementedError("write your pallas kernel here")


# speedup vs baseline: 82.2443x; 82.2443x over previous
import functools

import jax
import jax.numpy as jnp
from jax.experimental import pallas as pl
from jax.experimental.pallas import tpu as pltpu

def _topk_mask_kernel(x_ref, mask_ref, o_ref, *, k: int):
    int_min = jnp.int32(-(2**31))
    x = x_ref[0, 0] + mask_ref[0, 0]
    rows, n = x.shape

    y = jax.lax.bitcast_convert_type(x, jnp.int32)
    key = jnp.where(y < 0, y ^ jnp.int32(0x7FFFFFFF), y)

    def value_step(i, p):
        j = 31 - i
        bit = jax.lax.shift_left(jnp.int32(1), j)
        cand = p | bit
        cand_sig = cand ^ int_min
        cnt = jnp.sum((key >= cand_sig).astype(jnp.int32), axis=-1,
                      keepdims=True)
        return jnp.where(cnt >= k, cand, p)

    p0 = jnp.zeros((rows, 1), jnp.int32)
    p = jax.lax.fori_loop(0, 32, value_step, p0)
    thr = p ^ int_min

    gt = key > thr
    eq = key == thr
    cnt_gt = jnp.sum(gt.astype(jnp.int32), axis=-1, keepdims=True)
    need = k - cnt_gt

    idx = jax.lax.broadcasted_iota(jnp.int32, (rows, n), 1)

    def index_step(i, m):
        j = 10 - i
        bit = jax.lax.shift_left(jnp.int32(1), j)
        cand = m | (bit - 1)
        cnt = jnp.sum((eq & (idx <= cand)).astype(jnp.int32), axis=-1,
                      keepdims=True)
        return jnp.where(cnt >= need, m, m | bit)

    m0 = jnp.zeros((rows, 1), jnp.int32)
    m = jax.lax.fori_loop(0, 11, index_step, m0)

    kept = gt | (eq & (idx <= m))
    o_ref[0, 0] = jnp.where(kept, x, -jnp.inf)


@functools.partial(jax.jit, static_argnames=())
def kernel(attn_weights, attention_mask):
    bsz, num_heads, seq_len, n = attn_weights.shape
    k = max(1, int(0.5 * seq_len))
    k = min(k, seq_len)

    rows_per_block = 256
    grid = (bsz * num_heads, seq_len // rows_per_block)

    out = pl.pallas_call(
        functools.partial(_topk_mask_kernel, k=k),
        grid=grid,
        in_specs=[
            pl.BlockSpec((1, 1, rows_per_block, n),
                         lambda h, rb: (0, h, rb, 0)),
            pl.BlockSpec((1, 1, rows_per_block, n),
                         lambda h, rb: (0, 0, rb, 0)),
        ],
        out_specs=pl.BlockSpec((1, 1, rows_per_block, n),
                               lambda h, rb: (0, h, rb, 0)),
        out_shape=jax.ShapeDtypeStruct(attn_weights.shape, jnp.float32),
        compiler_params=pltpu.CompilerParams(
            dimension_semantics=("parallel", "parallel"),
        ),
    )(attn_weights.reshape(1, bsz * num_heads, seq_len, n),
      attention_mask)
    return out.reshape(bsz, num_heads, seq_len, n)


# speedup vs baseline: 107.7581x; 1.3102x over previous
import functools

import jax
import jax.numpy as jnp
from jax.experimental import pallas as pl
from jax.experimental.pallas import tpu as pltpu


def _count_ge(mask):
    m = mask.astype(jnp.int16)
    n = m.shape[-1]
    while n > 128:
        n //= 2
        m = m[:, :n] + m[:, n:]
    mb = m.astype(jnp.bfloat16)
    ones = jnp.ones((128, 128), jnp.bfloat16)
    cnt = jax.lax.dot_general(mb, ones, (((1,), (0,)), ((), ())),
                              preferred_element_type=jnp.float32)
    return cnt[:, :1]


def _topk_mask_kernel(x_ref, mask_ref, o_ref, *, k: int):
    x = x_ref[0, 0] + mask_ref[0, 0]
    rows, n = x.shape
    kf = jnp.float32(k)

    y = jax.lax.bitcast_convert_type(x, jnp.int32)
    key = jnp.where(y < 0, y ^ jnp.int32(0x7FFFFFFF), y)

    hi16 = jax.lax.shift_right_arithmetic(key, 16).astype(jnp.int16)
    lo_u = key & jnp.int32(0xFFFF)

    def hi_step(i, p):
        j = 15 - i
        cand = p | jax.lax.shift_left(jnp.int32(1), j)
        c16 = (cand - 32768).astype(jnp.int16)
        cnt = _count_ge(hi16 >= c16)
        return jnp.where(cnt >= kf, cand, p)

    p_hi = jax.lax.fori_loop(0, 16, hi_step, jnp.zeros((rows, 1), jnp.int32))
    thr_hi = (p_hi - 32768).astype(jnp.int16)

    hi_eq = hi16 == thr_hi
    cnt_gt_hi = _count_ge(hi16 > thr_hi)
    k_b = kf - cnt_gt_hi

    lo16 = jnp.where(hi_eq, (lo_u - 32768).astype(jnp.int16),
                     jnp.int16(-32768))

    def lo_step(i, p):
        j = 15 - i
        cand = p | jax.lax.shift_left(jnp.int32(1), j)
        c16 = (cand - 32768).astype(jnp.int16)
        cnt = _count_ge(lo16 >= c16)
        return jnp.where(cnt >= k_b, cand, p)

    p_lo = jax.lax.fori_loop(0, 16, lo_step, jnp.zeros((rows, 1), jnp.int32))
    thr_lo = (p_lo - 32768).astype(jnp.int16)

    gt = (hi16 > thr_hi) | (hi_eq & (lo16 > thr_lo))
    eq = hi_eq & (lo16 == thr_lo)
    cnt_gt = _count_ge(gt)
    need = kf - cnt_gt
    cnt_eq = _count_ge(eq)

    idx16 = jax.lax.broadcasted_iota(jnp.int32, (rows, n), 1)
    idxm = jnp.where(eq, (idx16 - 32768).astype(jnp.int16),
                     jnp.int16(32767))

    def any_conflict():
        return jnp.any(cnt_eq != need)

    def run_tie():
        def idx_step(i, m):
            j = 10 - i
            bit = jax.lax.shift_left(jnp.int32(1), j)
            cand = m | (bit - 1)
            c16 = (cand - 32768).astype(jnp.int16)
            cnt = _count_ge(idxm <= c16)
            return jnp.where(cnt >= need, m, m | bit)

        return jax.lax.fori_loop(0, 11, idx_step,
                                 jnp.zeros((rows, 1), jnp.int32))

    m_fin = jax.lax.cond(any_conflict(), run_tie,
                         lambda: jnp.full((rows, 1), n - 1, jnp.int32))
    m16 = (m_fin - 32768).astype(jnp.int16)

    kept = gt | (idxm <= m16)
    o_ref[0, 0] = jnp.where(kept, x, -jnp.inf)


def kernel(attn_weights, attention_mask):
    bsz, num_heads, seq_len, n = attn_weights.shape
    k = max(1, int(0.5 * seq_len))
    k = min(k, seq_len)

    rows_per_block = 256
    grid = (bsz * num_heads, seq_len // rows_per_block)

    out = pl.pallas_call(
        functools.partial(_topk_mask_kernel, k=k),
        grid=grid,
        in_specs=[
            pl.BlockSpec((1, 1, rows_per_block, n),
                         lambda h, rb: (0, h, rb, 0)),
            pl.BlockSpec((1, 1, rows_per_block, n),
                         lambda h, rb: (0, 0, rb, 0)),
        ],
        out_specs=pl.BlockSpec((1, 1, rows_per_block, n),
                               lambda h, rb: (0, h, rb, 0)),
        out_shape=jax.ShapeDtypeStruct(attn_weights.shape, jnp.float32),
        compiler_params=pltpu.CompilerParams(
            dimension_semantics=("parallel", "parallel"),
        ),
    )(attn_weights.reshape(1, bsz * num_heads, seq_len, n),
      attention_mask)
    return out.reshape(bsz, num_heads, seq_len, n)


# speedup vs baseline: 142.0499x; 1.3182x over previous
import functools

import jax
import jax.numpy as jnp
from jax.experimental import pallas as pl
from jax.experimental.pallas import tpu as pltpu


def _count_ge(mask):
    m = mask.astype(jnp.int16)
    n = m.shape[-1]
    while n > 128:
        n //= 2
        m = m[:, :n] + m[:, n:]
    mb = m.astype(jnp.bfloat16)
    ones = jnp.ones((128, 128), jnp.bfloat16)
    cnt = jax.lax.dot_general(mb, ones, (((1,), (0,)), ((), ())),
                              preferred_element_type=jnp.float32)
    return cnt[:, :1]


def _topk_mask_kernel(x_ref, mask_ref, o_ref, *, k: int):
    x = x_ref[0, 0] + mask_ref[0, 0]
    rows, n = x.shape
    half = rows // 2
    kf = jnp.float32(k)

    y = jax.lax.bitcast_convert_type(x, jnp.int32)
    key = jnp.where(y < 0, y ^ jnp.int32(0x7FFFFFFF), y)

    hi16 = jax.lax.shift_right_arithmetic(key, 16).astype(jnp.int16)
    lo_u = key & jnp.int32(0xFFFF)

    hi_a, hi_b = hi16[:half], hi16[half:]

    def hi_step(i, ps):
        pa, pb = ps
        j = 15 - i
        bit = jax.lax.shift_left(jnp.int32(1), j)
        ca, cb = pa | bit, pb | bit
        cnt_a = _count_ge(hi_a >= (ca - 32768).astype(jnp.int16))
        cnt_b = _count_ge(hi_b >= (cb - 32768).astype(jnp.int16))
        return (jnp.where(cnt_a >= kf, ca, pa),
                jnp.where(cnt_b >= kf, cb, pb))

    z = jnp.zeros((half, 1), jnp.int32)
    p_hi_a, p_hi_b = jax.lax.fori_loop(0, 16, hi_step, (z, z))
    p_hi = jnp.concatenate([p_hi_a, p_hi_b], axis=0)
    thr_hi = (p_hi - 32768).astype(jnp.int16)

    hi_eq = hi16 == thr_hi
    cnt_gt_hi = _count_ge(hi16 > thr_hi)
    k_b = kf - cnt_gt_hi

    lo16 = jnp.where(hi_eq, (lo_u - 32768).astype(jnp.int16),
                     jnp.int16(-32768))
    lo_a, lo_b = lo16[:half], lo16[half:]
    kb_a, kb_b = k_b[:half], k_b[half:]

    def lo_step(i, ps):
        pa, pb = ps
        j = 15 - i
        bit = jax.lax.shift_left(jnp.int32(1), j)
        ca, cb = pa | bit, pb | bit
        cnt_a = _count_ge(lo_a >= (ca - 32768).astype(jnp.int16))
        cnt_b = _count_ge(lo_b >= (cb - 32768).astype(jnp.int16))
        return (jnp.where(cnt_a >= kb_a, ca, pa),
                jnp.where(cnt_b >= kb_b, cb, pb))

    p_lo_a, p_lo_b = jax.lax.fori_loop(0, 16, lo_step, (z, z))
    p_lo = jnp.concatenate([p_lo_a, p_lo_b], axis=0)
    thr_lo = (p_lo - 32768).astype(jnp.int16)

    gt = (hi16 > thr_hi) | (hi_eq & (lo16 > thr_lo))
    eq = hi_eq & (lo16 == thr_lo)
    cnt_gt = _count_ge(gt)
    need = kf - cnt_gt
    cnt_eq = _count_ge(eq)

    idx16 = jax.lax.broadcasted_iota(jnp.int32, (rows, n), 1)
    idxm = jnp.where(eq, (idx16 - 32768).astype(jnp.int16),
                     jnp.int16(32767))
    ix_a, ix_b = idxm[:half], idxm[half:]
    nd_a, nd_b = need[:half], need[half:]

    def any_conflict():
        return jnp.any(cnt_eq != need)

    def run_tie():
        def idx_step(i, ms):
            ma, mb = ms
            j = 10 - i
            bit = jax.lax.shift_left(jnp.int32(1), j)
            ca, cb = ma | (bit - 1), mb | (bit - 1)
            cnt_a = _count_ge(ix_a <= (ca - 32768).astype(jnp.int16))
            cnt_b = _count_ge(ix_b <= (cb - 32768).astype(jnp.int16))
            return (jnp.where(cnt_a >= nd_a, ma, ma | bit),
                    jnp.where(cnt_b >= nd_b, mb, mb | bit))

        ma, mb = jax.lax.fori_loop(0, 11, idx_step, (z, z))
        return jnp.concatenate([ma, mb], axis=0)

    m_fin = jax.lax.cond(any_conflict(), run_tie,
                         lambda: jnp.full((rows, 1), n - 1, jnp.int32))
    m16 = (m_fin - 32768).astype(jnp.int16)

    kept = gt | (idxm <= m16)
    o_ref[0, 0] = jnp.where(kept, x, -jnp.inf)


def kernel(attn_weights, attention_mask):
    bsz, num_heads, seq_len, n = attn_weights.shape
    k = max(1, int(0.5 * seq_len))
    k = min(k, seq_len)

    rows_per_block = 512
    grid = (seq_len // rows_per_block, bsz * num_heads)

    out = pl.pallas_call(
        functools.partial(_topk_mask_kernel, k=k),
        grid=grid,
        in_specs=[
            pl.BlockSpec((1, 1, rows_per_block, n),
                         lambda rb, h: (0, h, rb, 0)),
            pl.BlockSpec((1, 1, rows_per_block, n),
                         lambda rb, h: (0, 0, rb, 0)),
        ],
        out_specs=pl.BlockSpec((1, 1, rows_per_block, n),
                               lambda rb, h: (0, h, rb, 0)),
        out_shape=jax.ShapeDtypeStruct(attn_weights.shape, jnp.float32),
        compiler_params=pltpu.CompilerParams(
            dimension_semantics=("parallel", "parallel"),
        ),
    )(attn_weights.reshape(1, bsz * num_heads, seq_len, n),
      attention_mask)
    return out.reshape(bsz, num_heads, seq_len, n)


# speedup vs baseline: 169.7047x; 1.1947x over previous
import functools

import jax
import jax.numpy as jnp
from jax.experimental import pallas as pl
from jax.experimental.pallas import tpu as pltpu


def _count_ge(mask):
    m = jnp.where(mask, jnp.bfloat16(1), jnp.bfloat16(0))
    n = m.shape[-1]
    while n > 128:
        n //= 2
        m = m[:, :n] + m[:, n:]
    ones = jnp.ones((128, 128), jnp.bfloat16)
    cnt = jax.lax.dot_general(m, ones, (((1,), (0,)), ((), ())),
                              preferred_element_type=jnp.float32)
    return cnt[:, :1]


def _topk_mask_kernel(x_ref, mask_ref, o_ref, *, k: int):
    x = x_ref[0, 0] + mask_ref[0, 0]
    rows, n = x.shape
    half = rows // 2
    kf = jnp.float32(k)

    y = jax.lax.bitcast_convert_type(x, jnp.int32)
    key = jnp.where(y < 0, y ^ jnp.int32(0x7FFFFFFF), y)

    hi16 = jax.lax.shift_right_arithmetic(key, 16).astype(jnp.int16)
    lo_s = (key ^ jnp.int32(0x8000)).astype(jnp.int16)

    hi_a, hi_b = hi16[:half], hi16[half:]

    def hi_step(i, st):
        pa, pb, ga, gb = st
        j = 15 - i
        bit = jax.lax.shift_left(jnp.int32(1), j)
        ca, cb = pa | bit, pb | bit
        cnt_a = _count_ge(hi_a >= (ca - 32768).astype(jnp.int16))
        cnt_b = _count_ge(hi_b >= (cb - 32768).astype(jnp.int16))
        ka, kb = cnt_a >= kf, cnt_b >= kf
        return (jnp.where(ka, ca, pa), jnp.where(kb, cb, pb),
                jnp.where(ka, cnt_a, ga), jnp.where(kb, cnt_b, gb))

    z = jnp.zeros((half, 1), jnp.int32)
    nf = jnp.full((half, 1), n, jnp.float32)
    p_hi_a, p_hi_b, cge_a, cge_b = jax.lax.fori_loop(
        0, 16, hi_step, (z, z, nf, nf))
    p_hi = jnp.concatenate([p_hi_a, p_hi_b], axis=0)
    thr_hi = (p_hi - 32768).astype(jnp.int16)

    hi_eq = hi16 == thr_hi
    cnt_gt_hi = _count_ge(hi16 > thr_hi)
    k_b = kf - cnt_gt_hi
    cnt_eq_hi = jnp.concatenate([cge_a, cge_b], axis=0) - cnt_gt_hi

    lo16 = jnp.where(hi_eq, lo_s, jnp.int16(-32768))
    lo_a, lo_b = lo16[:half], lo16[half:]
    kb_a, kb_b = k_b[:half], k_b[half:]
    ceq_a, ceq_b = cnt_eq_hi[:half], cnt_eq_hi[half:]

    def lo_step_factory(jbase):
        def lo_step(i, st):
            pa, pb, ga, gb = st
            j = jbase - i
            bit = jax.lax.shift_left(jnp.int32(1), j)
            ca, cb = pa | bit, pb | bit
            cnt_a = _count_ge(lo_a >= (ca - 32768).astype(jnp.int16))
            cnt_b = _count_ge(lo_b >= (cb - 32768).astype(jnp.int16))
            ka, kb = cnt_a >= kb_a, cnt_b >= kb_b
            return (jnp.where(ka, ca, pa), jnp.where(kb, cb, pb),
                    jnp.where(ka, cnt_a, ga), jnp.where(kb, cnt_b, gb))
        return lo_step

    st8 = jax.lax.fori_loop(0, 8, lo_step_factory(15),
                            (z, z, ceq_a, ceq_b))

    def run_lo8():
        return jax.lax.fori_loop(0, 8, lo_step_factory(7), st8)

    lo_conflict = jnp.any(st8[2] != kb_a) | jnp.any(st8[3] != kb_b)
    p_lo_a, p_lo_b, cbf_a, cbf_b = jax.lax.cond(
        lo_conflict, run_lo8, lambda: st8)
    p_lo = jnp.concatenate([p_lo_a, p_lo_b], axis=0)
    thr_lo = (p_lo - 32768).astype(jnp.int16)

    gt = (hi16 > thr_hi) | (lo16 > thr_lo)
    eq = hi_eq & (lo16 == thr_lo)

    cnt_ge_tot = cnt_gt_hi + jnp.concatenate([cbf_a, cbf_b], axis=0)
    tie_conflict = jnp.any(cnt_ge_tot != kf)

    idx16 = jax.lax.broadcasted_iota(jnp.int32, (rows, n), 1).astype(
        jnp.int16)

    def run_tie():
        cnt_gt = _count_ge(gt)
        need = kf - cnt_gt
        nd_a, nd_b = need[:half], need[half:]
        idxm = jnp.where(eq, idx16, jnp.int16(32767))
        ix_a, ix_b = idxm[:half], idxm[half:]

        def idx_step(i, ms):
            ma, mb = ms
            j = 10 - i
            bit = jax.lax.shift_left(jnp.int32(1), j)
            ca, cb = ma | (bit - 1), mb | (bit - 1)
            cnt_a = _count_ge(ix_a <= ca.astype(jnp.int16))
            cnt_b = _count_ge(ix_b <= cb.astype(jnp.int16))
            return (jnp.where(cnt_a >= nd_a, ma, ma | bit),
                    jnp.where(cnt_b >= nd_b, mb, mb | bit))

        return jax.lax.fori_loop(0, 11, idx_step, (z, z))

    nm1 = jnp.full((half, 1), n - 1, jnp.int32)
    m_a, m_b = jax.lax.cond(tie_conflict, run_tie, lambda: (nm1, nm1))
    m16 = jnp.concatenate([m_a, m_b], axis=0).astype(jnp.int16)

    kept = gt | (eq & (idx16 <= m16))
    o_ref[0, 0] = jnp.where(kept, x, -jnp.inf)


def kernel(attn_weights, attention_mask):
    bsz, num_heads, seq_len, n = attn_weights.shape
    k = max(1, int(0.5 * seq_len))
    k = min(k, seq_len)

    rows_per_block = 512
    grid = (seq_len // rows_per_block, bsz * num_heads)

    out = pl.pallas_call(
        functools.partial(_topk_mask_kernel, k=k),
        grid=grid,
        in_specs=[
            pl.BlockSpec((1, 1, rows_per_block, n),
                         lambda rb, h: (0, h, rb, 0)),
            pl.BlockSpec((1, 1, rows_per_block, n),
                         lambda rb, h: (0, 0, rb, 0)),
        ],
        out_specs=pl.BlockSpec((1, 1, rows_per_block, n),
                               lambda rb, h: (0, h, rb, 0)),
        out_shape=jax.ShapeDtypeStruct(attn_weights.shape, jnp.float32),
        compiler_params=pltpu.CompilerParams(
            dimension_semantics=("parallel", "parallel"),
        ),
    )(attn_weights.reshape(1, bsz * num_heads, seq_len, n),
      attention_mask)
    return out.reshape(bsz, num_heads, seq_len, n)
